# baseline (speedup 1.0000x reference)
import jax
import jax.numpy as jnp
from jax import lax
from jax.experimental import pallas as pl
from jax.experimental.pallas import tpu as pltpu
from jax.experimental.pallas import tpu_sc as plsc

_N = 10000
_E = 320000
_D = 128
_DE = 16

_NC = 2
_NS = 16

_CH = 128
_NROWS = _E // _CH
_NSUP = _NROWS // 8
_SPC = _NSUP // _NC
_STAIL = _NROWS - 8 * _NSUP
_CPT = -(-_SPC // _NS)
_RPT = 624
_TAIL = _N - _NS * _RPT


_BE = 2000


def _edge_mlp_body(ef_ref, w1_ref, b1_ref, w2_ref, b2_ref, o_ref):
    x = ef_ref[...]
    h1 = jnp.maximum(
        jnp.dot(x, w1_ref[...], preferred_element_type=jnp.float32) + b1_ref[...],
        0.0,
    )
    o_ref[...] = (
        jnp.dot(h1, w2_ref[...], preferred_element_type=jnp.float32) + b2_ref[...]
    )


def _edge_mlp(ef, We1, be1, We2, be2):
    grid = _E // _BE
    return pl.pallas_call(
        _edge_mlp_body,
        grid=(grid,),
        in_specs=[
            pl.BlockSpec((_BE, _DE), lambda i: (i, 0)),
            pl.BlockSpec((_DE, _D), lambda i: (0, 0)),
            pl.BlockSpec((1, _D), lambda i: (0, 0)),
            pl.BlockSpec((_D, _D), lambda i: (0, 0)),
            pl.BlockSpec((1, _D), lambda i: (0, 0)),
        ],
        out_specs=pl.BlockSpec((_BE, _D), lambda i: (i, 0)),
        out_shape=jax.ShapeDtypeStruct((_E, _D), jnp.float32),
    )(ef, We1, be1, We2, be2)


def _sc_layer_kernel(h_hbm, ei_hbm, e_hbm, out_hbm, src_v, dst_v, src_a, src_b,
                     dst_a, dst_b, e_v, rows_v, aggr_sh, sem_ga, sem_gb, sem_e,
                     sem_s):
    cid = lax.axis_index("c")
    sid = lax.axis_index("s")

    def _zero_row(j, carry):
        for k in range(_D // 16):
            e_v[j, pl.ds(k * 16, 16)] = jnp.zeros((16,), jnp.float32)
        return carry

    lax.fori_loop(0, _CH, _zero_row, 0)
    base_row = sid * _RPT
    for off in range(0, 512, _CH):
        pltpu.sync_copy(e_v.at[pl.ds(0, _CH)],
                        aggr_sh.at[pl.ds(base_row + off, _CH)])
    pltpu.sync_copy(e_v.at[pl.ds(0, _RPT - 512)],
                    aggr_sh.at[pl.ds(base_row + 512, _RPT - 512)])

    @pl.when(sid == _NS - 1)
    def _zero_tail():
        pltpu.sync_copy(e_v.at[pl.ds(0, _TAIL)],
                        aggr_sh.at[pl.ds(_NS * _RPT, _TAIL)])

    plsc.subcore_barrier()

    _HC = _CH // 2

    def _sub_chunk(g, j2):
        c = 8 * g + j2
        for t in range(4):
            sl16 = pl.ds(t * 16, 16)
            sh16 = pl.ds(_HC + t * 16, 16)
            src_a[sl16] = src_v[j2, sl16]
            src_b[sl16] = src_v[j2, sh16]
            dst_a[sl16] = dst_v[j2, sl16]
            dst_b[sl16] = dst_v[j2, sh16]
        de = pltpu.async_copy(e_hbm.at[pl.ds(_CH * c, _CH)], e_v, sem_e)
        da = pltpu.async_copy(h_hbm.at[src_a], rows_v.at[pl.ds(0, _HC)], sem_ga)
        db = pltpu.async_copy(h_hbm.at[src_b], rows_v.at[pl.ds(_HC, _HC)],
                              sem_gb)

        def _half(lo):
            @plsc.parallel_loop(lo, lo + _HC, unroll=4)
            def _row(j):
                for k2 in range(_D // 16):
                    sl = pl.ds(k2 * 16, 16)
                    rows_v[j, sl] = jnp.maximum(rows_v[j, sl] + e_v[j, sl], 0.0)

        da.wait()
        de.wait()
        _half(0)
        sa = pltpu.async_copy(rows_v.at[pl.ds(0, _HC)], aggr_sh.at[dst_a],
                              sem_s, add=True)
        db.wait()
        _half(_HC)
        sb = pltpu.async_copy(rows_v.at[pl.ds(_HC, _HC)], aggr_sh.at[dst_b],
                              sem_s, add=True)
        sa.wait()
        sb.wait()

    def _super(k, carry):
        s = sid + k * _NS

        @pl.when(s < _SPC)
        def _full():
            g = _NC * s + cid
            pltpu.sync_copy(ei_hbm.at[0, pl.ds(8 * g, 8)], src_v)
            pltpu.sync_copy(ei_hbm.at[1, pl.ds(8 * g, 8)], dst_v)
            for j2 in range(8):
                _sub_chunk(g, j2)

        return carry

    lax.fori_loop(0, _CPT, _super, 0)

    @pl.when((cid == 0) & (sid == 0))
    def _edge_tail():
        pltpu.sync_copy(ei_hbm.at[0, pl.ds(8 * _NSUP, _STAIL)],
                        src_v.at[pl.ds(0, _STAIL)])
        pltpu.sync_copy(ei_hbm.at[1, pl.ds(8 * _NSUP, _STAIL)],
                        dst_v.at[pl.ds(0, _STAIL)])
        for j2 in range(_STAIL):
            _sub_chunk(_NSUP, j2)

    plsc.subcore_barrier()

    pltpu.sync_copy(aggr_sh.at[pl.ds(base_row, _RPT)],
                    out_hbm.at[cid, pl.ds(base_row, _RPT)])

    @pl.when(sid == _NS - 1)
    def _write_tail():
        pltpu.sync_copy(aggr_sh.at[pl.ds(_NS * _RPT, _TAIL)],
                        out_hbm.at[cid, pl.ds(_NS * _RPT, _TAIL)])


def _sc_layer(h, ei3, e):
    mesh = plsc.VectorSubcoreMesh(core_axis_name="c", subcore_axis_name="s",
                                  num_cores=_NC, num_subcores=_NS)
    return pl.kernel(
        _sc_layer_kernel,
        out_type=jax.ShapeDtypeStruct((_NC, _N, _D), jnp.float32),
        mesh=mesh,
        scratch_types=[
            pltpu.VMEM((8, 128), jnp.int32),
            pltpu.VMEM((8, 128), jnp.int32),
            pltpu.VMEM((64,), jnp.int32),
            pltpu.VMEM((64,), jnp.int32),
            pltpu.VMEM((64,), jnp.int32),
            pltpu.VMEM((64,), jnp.int32),
            pltpu.VMEM((_CH, _D), jnp.float32),
            pltpu.VMEM((_CH, _D), jnp.float32),
            pltpu.VMEM_SHARED((_N, _D), jnp.float32),
            pltpu.SemaphoreType.DMA,
            pltpu.SemaphoreType.DMA,
            pltpu.SemaphoreType.DMA,
            pltpu.SemaphoreType.DMA,
        ],
    )(h, ei3, e)


_BN = 2000


def _node_body(h_ref, a_ref, w_ref, b_ref, o_ref):
    x = h_ref[...] + a_ref[0] + a_ref[1]
    o_ref[...] = jnp.maximum(
        jnp.dot(x, w_ref[...], preferred_element_type=jnp.float32) + b_ref[...],
        0.0,
    )


def _node_update(h, parts, W, b):
    grid = _N // _BN
    return pl.pallas_call(
        _node_body,
        grid=(grid,),
        in_specs=[
            pl.BlockSpec((_BN, _D), lambda i: (i, 0)),
            pl.BlockSpec((_NC, _BN, _D), lambda i: (0, i, 0)),
            pl.BlockSpec((_D, _D), lambda i: (0, 0)),
            pl.BlockSpec((1, _D), lambda i: (0, 0)),
        ],
        out_specs=pl.BlockSpec((_BN, _D), lambda i: (i, 0)),
        out_shape=jax.ShapeDtypeStruct((_N, _D), jnp.float32),
    )(h, parts, W, b)


def kernel(node_feats, edge_feats, edge_index, We1, be1, We2, be2, W0, b0, W1, b1):
    e = _edge_mlp(edge_feats, We1, be1.reshape(1, _D), We2, be2.reshape(1, _D))
    ei3 = edge_index.reshape(2, _E // 128, 128)
    h = node_feats
    for (W, b) in ((W0, b0), (W1, b1)):
        parts = _sc_layer(h, ei3, e)
        h = _node_update(h, parts, W, b.reshape(1, _D))
    return h


# speedup vs baseline: 1.2411x; 1.2411x over previous
import jax
import jax.numpy as jnp
from jax import lax
from jax.experimental import pallas as pl
from jax.experimental.pallas import tpu as pltpu
from jax.experimental.pallas import tpu_sc as plsc

_N = 10000
_E = 320000
_D = 128
_DE = 16

_NC = 2
_NS = 16

_CH = 128
_NROWS = _E // _CH
_NSUP = _NROWS // 8
_SPC = _NSUP // _NC
_STAIL = _NROWS - 8 * _NSUP
_CPT = -(-_SPC // _NS)
_RPT = 624
_TAIL = _N - _NS * _RPT


_BE = 2000


def _edge_mlp_body(ef_ref, w1_ref, b1_ref, w2_ref, b2_ref, o_ref):
    x = ef_ref[...]
    h1 = jnp.maximum(
        jnp.dot(x, w1_ref[...], preferred_element_type=jnp.float32) + b1_ref[...],
        0.0,
    )
    o_ref[...] = (
        jnp.dot(h1, w2_ref[...], preferred_element_type=jnp.float32) + b2_ref[...]
    )


def _edge_mlp(ef, We1, be1, We2, be2):
    grid = _E // _BE
    return pl.pallas_call(
        _edge_mlp_body,
        grid=(grid,),
        in_specs=[
            pl.BlockSpec((_BE, _DE), lambda i: (i, 0)),
            pl.BlockSpec((_DE, _D), lambda i: (0, 0)),
            pl.BlockSpec((1, _D), lambda i: (0, 0)),
            pl.BlockSpec((_D, _D), lambda i: (0, 0)),
            pl.BlockSpec((1, _D), lambda i: (0, 0)),
        ],
        out_specs=pl.BlockSpec((_BE, _D), lambda i: (i, 0)),
        out_shape=jax.ShapeDtypeStruct((_E, _D), jnp.float32),
    )(ef, We1, be1, We2, be2)


def _sc_layer_kernel(h_hbm, ei_hbm, e_hbm, out_hbm, src_v, dst_v, src_a, src_b,
                     dst_a, dst_b, e_v, rows_v, aggr_sh, sem_ga, sem_gb, sem_e,
                     sem_s):
    cid = lax.axis_index("c")
    sid = lax.axis_index("s")

    def _zero_row(j, carry):
        for k in range(_D // 16):
            e_v[j, pl.ds(k * 16, 16)] = jnp.zeros((16,), jnp.float32)
        return carry

    lax.fori_loop(0, _CH, _zero_row, 0)
    base_row = sid * _RPT
    for off in range(0, 512, _CH):
        pltpu.sync_copy(e_v.at[pl.ds(0, _CH)],
                        aggr_sh.at[pl.ds(base_row + off, _CH)])
    pltpu.sync_copy(e_v.at[pl.ds(0, _RPT - 512)],
                    aggr_sh.at[pl.ds(base_row + 512, _RPT - 512)])

    @pl.when(sid == _NS - 1)
    def _zero_tail():
        pltpu.sync_copy(e_v.at[pl.ds(0, _TAIL)],
                        aggr_sh.at[pl.ds(_NS * _RPT, _TAIL)])

    plsc.subcore_barrier()

    _HC = _CH // 2
    src_p = (src_a, src_b)
    dst_p = (dst_a, dst_b)
    sem_g = (sem_ga, sem_gb)

    def _bounce(h):
        p = h % 2
        j2 = h // 2
        for t in range(4):
            sl16 = pl.ds(t * 16, 16)
            shalf = pl.ds(p * _HC + t * 16, 16)
            src_p[p][sl16] = src_v[j2, shalf]
            dst_p[p][sl16] = dst_v[j2, shalf]

    def _issue(g, h):
        p = h % 2
        row0 = p * _HC
        dg = pltpu.async_copy(h_hbm.at[src_p[p]],
                              rows_v.at[pl.ds(row0, _HC)], sem_g[p])
        de = pltpu.async_copy(e_hbm.at[pl.ds(1024 * g + 64 * h, _HC)],
                              e_v.at[pl.ds(row0, _HC)], sem_e)
        return dg, de

    def _run_super(g, nhalves):
        _bounce(0)
        descs = _issue(g, 0)
        prev_scatter = None
        for h in range(nhalves):
            p = h % 2
            row0 = p * _HC
            if h + 1 < nhalves:
                if prev_scatter is not None:
                    prev_scatter.wait()
                    prev_scatter = None
                _bounce(h + 1)
                next_descs = _issue(g, h + 1)
            descs[0].wait()
            descs[1].wait()

            def _row(j, carry2):
                for k2 in range(_D // 16):
                    sl = pl.ds(k2 * 16, 16)
                    rows_v[j, sl] = jnp.maximum(rows_v[j, sl] + e_v[j, sl], 0.0)
                return carry2

            lax.fori_loop(row0, row0 + _HC, _row, 0)
            if prev_scatter is not None:
                prev_scatter.wait()
            prev_scatter = pltpu.async_copy(rows_v.at[pl.ds(row0, _HC)],
                                            aggr_sh.at[dst_p[p]], sem_s,
                                            add=True)
            if h + 1 < nhalves:
                descs = next_descs
        prev_scatter.wait()

    def _super(k, carry):
        s = sid + k * _NS

        @pl.when(s < _SPC)
        def _full():
            g = _NC * s + cid
            pltpu.sync_copy(ei_hbm.at[0, pl.ds(8 * g, 8)], src_v)
            pltpu.sync_copy(ei_hbm.at[1, pl.ds(8 * g, 8)], dst_v)
            _run_super(g, 16)

        return carry

    lax.fori_loop(0, _CPT, _super, 0)

    @pl.when((cid == 0) & (sid == 0))
    def _edge_tail():
        pltpu.sync_copy(ei_hbm.at[0, pl.ds(8 * _NSUP, _STAIL)],
                        src_v.at[pl.ds(0, _STAIL)])
        pltpu.sync_copy(ei_hbm.at[1, pl.ds(8 * _NSUP, _STAIL)],
                        dst_v.at[pl.ds(0, _STAIL)])
        _run_super(_NSUP, 2 * _STAIL)

    plsc.subcore_barrier()

    pltpu.sync_copy(aggr_sh.at[pl.ds(base_row, _RPT)],
                    out_hbm.at[cid, pl.ds(base_row, _RPT)])

    @pl.when(sid == _NS - 1)
    def _write_tail():
        pltpu.sync_copy(aggr_sh.at[pl.ds(_NS * _RPT, _TAIL)],
                        out_hbm.at[cid, pl.ds(_NS * _RPT, _TAIL)])


def _sc_layer(h, ei3, e):
    mesh = plsc.VectorSubcoreMesh(core_axis_name="c", subcore_axis_name="s",
                                  num_cores=_NC, num_subcores=_NS)
    return pl.kernel(
        _sc_layer_kernel,
        out_type=jax.ShapeDtypeStruct((_NC, _N, _D), jnp.float32),
        mesh=mesh,
        scratch_types=[
            pltpu.VMEM((8, 128), jnp.int32),
            pltpu.VMEM((8, 128), jnp.int32),
            pltpu.VMEM((64,), jnp.int32),
            pltpu.VMEM((64,), jnp.int32),
            pltpu.VMEM((64,), jnp.int32),
            pltpu.VMEM((64,), jnp.int32),
            pltpu.VMEM((_CH, _D), jnp.float32),
            pltpu.VMEM((_CH, _D), jnp.float32),
            pltpu.VMEM_SHARED((_N, _D), jnp.float32),
            pltpu.SemaphoreType.DMA,
            pltpu.SemaphoreType.DMA,
            pltpu.SemaphoreType.DMA,
            pltpu.SemaphoreType.DMA,
        ],
    )(h, ei3, e)


_BN = 2000


def _node_body(h_ref, a_ref, w_ref, b_ref, o_ref):
    x = h_ref[...] + a_ref[0] + a_ref[1]
    o_ref[...] = jnp.maximum(
        jnp.dot(x, w_ref[...], preferred_element_type=jnp.float32) + b_ref[...],
        0.0,
    )


def _node_update(h, parts, W, b):
    grid = _N // _BN
    return pl.pallas_call(
        _node_body,
        grid=(grid,),
        in_specs=[
            pl.BlockSpec((_BN, _D), lambda i: (i, 0)),
            pl.BlockSpec((_NC, _BN, _D), lambda i: (0, i, 0)),
            pl.BlockSpec((_D, _D), lambda i: (0, 0)),
            pl.BlockSpec((1, _D), lambda i: (0, 0)),
        ],
        out_specs=pl.BlockSpec((_BN, _D), lambda i: (i, 0)),
        out_shape=jax.ShapeDtypeStruct((_N, _D), jnp.float32),
    )(h, parts, W, b)


def kernel(node_feats, edge_feats, edge_index, We1, be1, We2, be2, W0, b0, W1, b1):
    e = _edge_mlp(edge_feats, We1, be1.reshape(1, _D), We2, be2.reshape(1, _D))
    ei3 = edge_index.reshape(2, _E // 128, 128)
    h = node_feats
    for (W, b) in ((W0, b0), (W1, b1)):
        parts = _sc_layer(h, ei3, e)
        h = _node_update(h, parts, W, b.reshape(1, _D))
    return h
